# Initial kernel scaffold; baseline (speedup 1.0000x reference)
import jax
import jax.numpy as jnp
from jax.experimental import pallas as pl


def kernel(x, edge_index, edge_index_id, diags, is_null_centrality_mask, m1, m2, m3, e1, e2, e3, a, W0, b0, W1, b1, W2, b2):
    raise NotImplementedError("write your pallas kernel here")


# speedup vs baseline: 107.2617x; 107.2617x over previous
import functools

import jax
import jax.numpy as jnp
from jax import lax
from jax.experimental import pallas as pl
from jax.experimental.pallas import tpu as pltpu
from jax.experimental.pallas import tpu_sc as plsc

N = 10000
D = 128
E = 320000
E_ID = 330000
E_TOT = E + E_ID
NC = 2
NS = 16
NW = NC * NS
CHUNK = 128
EP = ((E_TOT + NW * CHUNK - 1) // (NW * CHUNK)) * (NW * CHUNK)
CPT = EP // (NW * CHUNK)
NPAD = 10240
NACC = 10240
RPT = NACC // NS
ZROWS = 128

_MESH = plsc.VectorSubcoreMesh(
    core_axis_name="c", subcore_axis_name="s", num_cores=NC, num_subcores=NS)


def _pow_body(d_ref, e_ref, o_ref):
    logd = jnp.log(d_ref[...])
    for j in range(3):
        o_ref[j] = jnp.exp(e_ref[j] * logd)


def _pow_tables(diags_p, evec):
    return pl.pallas_call(
        _pow_body,
        out_shape=jax.ShapeDtypeStruct((3, NPAD // 128, 128), jnp.float32),
    )(diags_p, evec)


def _w_body(d1_hbm, d2_hbm, d3_hbm, row_hbm, col_hbm, msk_hbm, sv_hbm, w_hbm,
            d1v, d2v, d3v, svv, ir, ic, mb, wb):
    c = lax.axis_index("c")
    s = lax.axis_index("s")
    wid = s * NC + c
    pltpu.sync_copy(d1_hbm, d1v)
    pltpu.sync_copy(d2_hbm, d2v)
    pltpu.sync_copy(d3_hbm, d3v)
    pltpu.sync_copy(sv_hbm, svv)
    m2 = svv[pl.ds(0, 16)]
    m1 = svv[pl.ds(16, 16)]
    m2a = svv[pl.ds(32, 16)]
    m3 = svv[pl.ds(48, 16)]
    lane = lax.iota(jnp.int32, 16)

    def chunk(t, _):
        e0 = (wid * CPT + t) * CHUNK
        pltpu.sync_copy(row_hbm.at[pl.ds(e0, CHUNK)], ir)
        pltpu.sync_copy(col_hbm.at[pl.ds(e0, CHUNK)], ic)
        pltpu.sync_copy(msk_hbm.at[pl.ds(e0, CHUNK)], mb)
        for i in range(CHUNK // 16):
            r16 = ir[pl.ds(i * 16, 16)]
            c16 = ic[pl.ds(i * 16, 16)]
            nm = 1.0 - mb[pl.ds(i * 16, 16)]
            d1r = plsc.load_gather(d1v, [r16])
            d2r = plsc.load_gather(d2v, [r16])
            d3c = plsc.load_gather(d3v, [c16])
            prod = d2r * d3c
            g1 = m2 * prod
            g2 = (m1 * d1r + m2a * prod) * nm + m3
            gi = e0 + i * 16 + lane
            w16 = jnp.where(gi < E, g1, jnp.where(gi < E_TOT, g2, 0.0))
            wb[pl.ds(i * 16, 16)] = w16
        pltpu.sync_copy(wb, w_hbm.at[pl.ds(e0, CHUNK)])
        return jnp.int32(0)

    lax.fori_loop(jnp.int32(0), jnp.int32(CPT), chunk, jnp.int32(0))


_w_kernel = functools.partial(
    pl.kernel,
    out_type=jax.ShapeDtypeStruct((EP,), jnp.float32),
    mesh=_MESH,
    compiler_params=pltpu.CompilerParams(needs_layout_passes=False),
    scratch_types=[
        pltpu.VMEM((NPAD,), jnp.float32),
        pltpu.VMEM((NPAD,), jnp.float32),
        pltpu.VMEM((NPAD,), jnp.float32),
        pltpu.VMEM((64,), jnp.float32),
        pltpu.VMEM((CHUNK,), jnp.int32),
        pltpu.VMEM((CHUNK,), jnp.int32),
        pltpu.VMEM((CHUNK,), jnp.float32),
        pltpu.VMEM((CHUNK,), jnp.float32),
    ],
)(_w_body)


def _agg_body(h_hbm, src_hbm, dst_hbm, w_hbm, out_hbm,
              idx_s, idx_d, wbuf, rows, stage, acc, sem):
    c = lax.axis_index("c")
    s = lax.axis_index("s")
    wid = s * NC + c
    zero16 = jnp.zeros((16,), jnp.float32)

    def zstage(i, _):
        for dd in range(D // 16):
            stage[i, pl.ds(dd * 16, 16)] = zero16
        return jnp.int32(0)

    lax.fori_loop(jnp.int32(0), jnp.int32(ZROWS), zstage, jnp.int32(0))
    base_row = s * RPT

    def zacc(z, _):
        pltpu.sync_copy(stage, acc.at[pl.ds(base_row + z * ZROWS, ZROWS)])
        return jnp.int32(0)

    lax.fori_loop(jnp.int32(0), jnp.int32(RPT // ZROWS), zacc, jnp.int32(0))
    plsc.subcore_barrier()

    def chunk(t, _):
        e0 = (wid * CPT + t) * CHUNK
        pltpu.sync_copy(src_hbm.at[pl.ds(e0, CHUNK)], idx_s)
        pltpu.sync_copy(dst_hbm.at[pl.ds(e0, CHUNK)], idx_d)
        pltpu.sync_copy(w_hbm.at[pl.ds(e0, CHUNK)], wbuf)
        pltpu.async_copy(h_hbm.at[idx_s], rows, sem).wait()

        def scale(g, _):
            ws = plsc.load_gather(wbuf, [jnp.full((16,), g, jnp.int32)])
            for dd in range(D // 16):
                rows[g, pl.ds(dd * 16, 16)] = rows[g, pl.ds(dd * 16, 16)] * ws
            return jnp.int32(0)

        lax.fori_loop(jnp.int32(0), jnp.int32(CHUNK), scale, jnp.int32(0))
        pltpu.sync_copy(rows, acc.at[idx_d], add=True)
        return jnp.int32(0)

    lax.fori_loop(jnp.int32(0), jnp.int32(CPT), chunk, jnp.int32(0))
    plsc.subcore_barrier()

    def cout(z, _):
        r0 = base_row + z * ZROWS
        pltpu.sync_copy(acc.at[pl.ds(r0, ZROWS)], stage)
        pltpu.sync_copy(stage, out_hbm.at[c, pl.ds(r0, ZROWS)])
        return jnp.int32(0)

    lax.fori_loop(jnp.int32(0), jnp.int32(RPT // ZROWS), cout, jnp.int32(0))


_agg_kernel = functools.partial(
    pl.kernel,
    out_type=jax.ShapeDtypeStruct((NC, NACC, D), jnp.float32),
    mesh=_MESH,
    compiler_params=pltpu.CompilerParams(needs_layout_passes=False),
    scratch_types=[
        pltpu.VMEM((CHUNK,), jnp.int32),
        pltpu.VMEM((CHUNK,), jnp.int32),
        pltpu.VMEM((CHUNK,), jnp.float32),
        pltpu.VMEM((CHUNK, D), jnp.float32),
        pltpu.VMEM((ZROWS, D), jnp.float32),
        pltpu.VMEM_SHARED((NACC, D), jnp.float32),
        pltpu.SemaphoreType.DMA,
    ],
)(_agg_body)


def _layer_body(p_ref, w_ref, b_ref, o_ref, *, last):
    ps = p_ref[0] + p_ref[1]
    h = lax.dot_general(ps, w_ref[...], (((1,), (0,)), ((), ())),
                        precision=lax.Precision.HIGHEST,
                        preferred_element_type=jnp.float32)
    h = h + 2.0 * b_ref[0]
    if last:
        m = jnp.max(h, axis=1, keepdims=True)
        h = (h - m) - jnp.log(jnp.sum(jnp.exp(h - m), axis=1, keepdims=True))
    else:
        h = jnp.maximum(h, 0.0)
    o_ref[...] = h


_BM = 1000


def _layer(parts, W, b, last):
    return pl.pallas_call(
        functools.partial(_layer_body, last=last),
        grid=(N // _BM,),
        in_specs=[
            pl.BlockSpec((NC, _BM, D), lambda i: (i * 0, i, i * 0)),
            pl.BlockSpec((D, D), lambda i: (i * 0, i * 0)),
            pl.BlockSpec((1, D), lambda i: (i * 0, i * 0)),
        ],
        out_specs=pl.BlockSpec((_BM, D), lambda i: (i, i * 0)),
        out_shape=jax.ShapeDtypeStruct((N, D), jnp.float32),
    )(parts, W, b)


def kernel(x, edge_index, edge_index_id, diags, is_null_centrality_mask,
           m1, m2, m3, e1, e2, e3, a, W0, b0, W1, b1, W2, b2):
    W0, W1, W2 = (w.astype(jnp.float32) for w in (W0, W1, W2))
    b0, b1, b2 = (b.astype(jnp.float32) for b in (b0, b1, b2))
    src = jnp.concatenate([edge_index[0], edge_index_id[0]]).astype(jnp.int32)
    dst = jnp.concatenate([edge_index[1], edge_index_id[1]]).astype(jnp.int32)
    pad = EP - E_TOT
    src = jnp.pad(src, (0, pad))
    dst = jnp.pad(dst, (0, pad))
    msk = jnp.pad(is_null_centrality_mask.astype(jnp.float32), (E, pad))
    diags_p = jnp.pad(diags, (0, NPAD - N), constant_values=1.0)
    diags_p = diags_p.reshape(NPAD // 128, 128)
    evec = jnp.stack([jnp.broadcast_to(e1, (128,)),
                      jnp.broadcast_to(e2, (128,)),
                      jnp.broadcast_to(e3, (128,))])
    svec = jnp.concatenate([jnp.broadcast_to(m2, (16,)),
                            jnp.broadcast_to(m1, (16,)),
                            jnp.broadcast_to(m2 * a, (16,)),
                            jnp.broadcast_to(m3, (16,))])

    d_tab = _pow_tables(diags_p, evec).reshape(3, NPAD)
    w_all = _w_kernel(d_tab[0], d_tab[1], d_tab[2], src, dst, msk, svec)

    h = x
    for W, b, last in ((W0, b0, False), (W1, b1, False), (W2, b2, True)):
        parts = _agg_kernel(h, src, dst, w_all)
        h = _layer(parts, W, b.reshape(1, D), last)
    return h.astype(jnp.float64)
